# baseline (speedup 1.0000x reference)
import jax
import jax.numpy as jnp
from jax import lax
from jax.experimental import pallas as pl
from jax.experimental.pallas import tpu as pltpu
from jax.experimental.pallas import tpu_sc as plsc

TOTAL = 32768
NBINS = 16
NS = 16
LANES = 16
SEARCH_STEPS = 16


def _sc_body(slot_hbm, nt_hbm, out_hbm, data_v, nt_v, hist_v, shift_v,
             sem_a, sem_b):
    sid = lax.axis_index("s")

    @pl.when(sid == 0)
    def _():
        cp_a = pltpu.make_async_copy(slot_hbm, data_v, sem_a)
        cp_b = pltpu.make_async_copy(nt_hbm, nt_v, sem_b)
        cp_a.start()
        cp_b.start()
        cp_a.wait()

        zeros = jnp.zeros((LANES,), jnp.int32)
        lane_iota = lax.iota(jnp.int32, LANES)

        lo = zeros
        hi = jnp.full((LANES,), TOTAL, jnp.int32)
        cap = jnp.full((LANES,), TOTAL - 1, jnp.int32)
        target = lane_iota + 1
        for _ in range(SEARCH_STEPS):
            mid = jnp.minimum((lo + hi) >> 1, cap)
            c = plsc.load_gather(data_v, [mid])
            pred = c < target
            lo = jnp.where(pred, mid + 1, lo)
            hi = jnp.where(pred, hi, mid)
        glb_hi = lo
        shift_v[pl.ds(0, LANES)] = zeros
        plsc.store_scatter(shift_v, [lane_iota + 1], glb_hi)
        glb_lo = shift_v[pl.ds(0, LANES)]
        cp_b.wait()
        nt_vec = nt_v[...]
        hist_v[...] = (jnp.minimum(glb_hi, nt_vec)
                       - jnp.minimum(glb_lo, nt_vec))
        pltpu.sync_copy(hist_v, out_hbm)


@jax.jit
def _counts_sc(slot_ids, nt_vec):
    mesh = plsc.VectorSubcoreMesh(
        core_axis_name="c", subcore_axis_name="s", num_cores=1,
        num_subcores=NS)
    return pl.kernel(
        _sc_body,
        out_type=jax.ShapeDtypeStruct((NBINS,), jnp.int32),
        mesh=mesh,
        scratch_types=[
            pltpu.VMEM((TOTAL,), jnp.int32),
            pltpu.VMEM((LANES,), jnp.int32),
            pltpu.VMEM((NBINS,), jnp.int32),
            pltpu.VMEM((LANES + 1,), jnp.int32),
            pltpu.SemaphoreType.DMA,
            pltpu.SemaphoreType.DMA,
        ],
        compiler_params=pltpu.CompilerParams(needs_layout_passes=False),
    )(slot_ids, nt_vec)


def kernel(tokens, slot_ids, pos_ids, num_tokens, max_slots):
    nt_vec = jnp.full((LANES,), num_tokens, dtype=jnp.int32)
    return _counts_sc(slot_ids, nt_vec)


# speedup vs baseline: 1.0382x; 1.0382x over previous
import jax
import jax.numpy as jnp
from jax import lax
from jax.experimental import pallas as pl
from jax.experimental.pallas import tpu as pltpu
from jax.experimental.pallas import tpu_sc as plsc

TOTAL = 32768
NBINS = 16
NS = 16
LANES = 16
CHUNK = TOTAL // NS
SEARCH_STEPS = 12


def _sc_body(slot_hbm, nt_hbm, out_hbm, chunk_v, nt_v, hist_v, rows_v,
             rows_l, shift_v, sem_a, sem_b):
    sid = lax.axis_index("s")
    base = sid * CHUNK

    cp_chunk = pltpu.make_async_copy(
        slot_hbm.at[pl.ds(base, CHUNK)], chunk_v, sem_a)
    cp_chunk.start()

    @pl.when(sid == 0)
    def _():
        pltpu.make_async_copy(nt_hbm, nt_v, sem_b).start()

    cp_chunk.wait()

    zeros = jnp.zeros((LANES,), jnp.int32)
    lane_iota = lax.iota(jnp.int32, LANES)

    lo = zeros
    hi = jnp.full((LANES,), CHUNK, jnp.int32)
    target = lane_iota + 1
    cap = jnp.full((LANES,), CHUNK - 1, jnp.int32)
    for _ in range(SEARCH_STEPS):
        mid = jnp.minimum((lo + hi) >> 1, cap)
        c = plsc.load_gather(chunk_v, [mid])
        pred = c < target
        lo = jnp.where(pred, mid + 1, lo)
        hi = jnp.where(pred, hi, mid)

    @pl.when(sid != 0)
    def _():
        hist_v[...] = lo
        pltpu.sync_copy(hist_v, rows_v.at[pl.ds(sid * NBINS, NBINS)])

    plsc.subcore_barrier()

    @pl.when(sid == 0)
    def _():
        pltpu.sync_copy(rows_v.at[pl.ds(NBINS, (NS - 1) * NBINS)],
                        rows_l)
        glb_hi = lo
        for r in range(NS - 1):
            glb_hi = glb_hi + rows_l[pl.ds(r * NBINS, NBINS)]
        shift_v[pl.ds(0, LANES)] = zeros
        plsc.store_scatter(shift_v, [lane_iota + 1], glb_hi)
        glb_lo = shift_v[pl.ds(0, LANES)]
        pltpu.make_async_copy(nt_hbm, nt_v, sem_b).wait()
        nt_vec = plsc.load_gather(nt_v, [zeros])
        hist_v[...] = (jnp.minimum(glb_hi, nt_vec)
                       - jnp.minimum(glb_lo, nt_vec))
        pltpu.sync_copy(hist_v, out_hbm)


@jax.jit
def _counts_sc(slot_ids, nt_vec):
    mesh = plsc.VectorSubcoreMesh(
        core_axis_name="c", subcore_axis_name="s", num_cores=1,
        num_subcores=NS)
    return pl.kernel(
        _sc_body,
        out_type=jax.ShapeDtypeStruct((NBINS,), jnp.int32),
        mesh=mesh,
        scratch_types=[
            pltpu.VMEM((CHUNK,), jnp.int32),
            pltpu.VMEM((1,), jnp.int32),
            pltpu.VMEM((NBINS,), jnp.int32),
            pltpu.VMEM_SHARED((NS * NBINS,), jnp.int32),
            pltpu.VMEM(((NS - 1) * NBINS,), jnp.int32),
            pltpu.VMEM((LANES + 1,), jnp.int32),
            pltpu.SemaphoreType.DMA,
            pltpu.SemaphoreType.DMA,
        ],
        compiler_params=pltpu.CompilerParams(needs_layout_passes=False),
    )(slot_ids, nt_vec)


def kernel(tokens, slot_ids, pos_ids, num_tokens, max_slots):
    nt_arr = jnp.reshape(num_tokens.astype(jnp.int32), (1,))
    return _counts_sc(slot_ids, nt_arr)


# speedup vs baseline: 1.0445x; 1.0061x over previous
import jax
import jax.numpy as jnp
from jax import lax
from jax.experimental import pallas as pl
from jax.experimental.pallas import tpu as pltpu
from jax.experimental.pallas import tpu_sc as plsc

TOTAL = 32768
NBINS = 16
NS = 16
LANES = 16
CHUNK = TOTAL // NS
SEARCH_STEPS = 12


def _sc_body(slot_hbm, nt_hbm, out_hbm, chunk_v, nt_v, hist_v, rows_v,
             rows_l, shift_v, sem_a, sem_b):
    sid = lax.axis_index("s")
    base = sid * CHUNK

    cp_chunk = pltpu.make_async_copy(
        slot_hbm.at[pl.ds(base, CHUNK)], chunk_v, sem_a)
    cp_chunk.start()

    @pl.when(sid == 0)
    def _():
        pltpu.make_async_copy(nt_hbm, nt_v, sem_b).start()

    cp_chunk.wait()

    zeros = jnp.zeros((LANES,), jnp.int32)
    lane_iota = lax.iota(jnp.int32, LANES)

    lo = zeros
    hi = jnp.full((LANES,), CHUNK, jnp.int32)
    target = lane_iota + 1
    cap = jnp.full((LANES,), CHUNK - 1, jnp.int32)
    for _ in range(SEARCH_STEPS):
        mid = jnp.minimum((lo + hi) >> 1, cap)
        c = plsc.load_gather(chunk_v, [mid])
        pred = c < target
        lo = jnp.where(pred, mid + 1, lo)
        hi = jnp.where(pred, hi, mid)

    @pl.when(sid != 0)
    def _():
        hist_v[...] = lo
        pltpu.sync_copy(hist_v, rows_v.at[pl.ds(sid * NBINS, NBINS)])

    plsc.subcore_barrier()

    @pl.when(sid == 0)
    def _():
        pltpu.sync_copy(rows_v.at[pl.ds(NBINS, (NS - 1) * NBINS)], rows_l)
        glb_hi = lo
        for r in range(NS - 1):
            glb_hi = glb_hi + rows_l[pl.ds(r * NBINS, NBINS)]
        shift_v[pl.ds(0, LANES)] = zeros
        plsc.store_scatter(shift_v, [lane_iota + 1], glb_hi)
        glb_lo = shift_v[pl.ds(0, LANES)]
        pltpu.make_async_copy(nt_hbm, nt_v, sem_b).wait()
        nt_vec = nt_v[...]
        hist_v[...] = (jnp.minimum(glb_hi, nt_vec)
                       - jnp.minimum(glb_lo, nt_vec))
        pltpu.sync_copy(hist_v, out_hbm)


@jax.jit
def _counts_sc(slot_ids, nt_vec):
    mesh = plsc.VectorSubcoreMesh(
        core_axis_name="c", subcore_axis_name="s", num_cores=1,
        num_subcores=NS)
    return pl.kernel(
        _sc_body,
        out_type=jax.ShapeDtypeStruct((NBINS,), jnp.int32),
        mesh=mesh,
        scratch_types=[
            pltpu.VMEM((CHUNK,), jnp.int32),
            pltpu.VMEM((LANES,), jnp.int32),
            pltpu.VMEM((NBINS,), jnp.int32),
            pltpu.VMEM_SHARED((NS * NBINS,), jnp.int32),
            pltpu.VMEM(((NS - 1) * NBINS,), jnp.int32),
            pltpu.VMEM((LANES + 1,), jnp.int32),
            pltpu.SemaphoreType.DMA,
            pltpu.SemaphoreType.DMA,
        ],
        compiler_params=pltpu.CompilerParams(needs_layout_passes=False),
    )(slot_ids, nt_vec)


def kernel(tokens, slot_ids, pos_ids, num_tokens, max_slots):
    nt_vec = jnp.full((LANES,), num_tokens, dtype=jnp.int32)
    return _counts_sc(slot_ids, nt_vec)


# speedup vs baseline: 1.0463x; 1.0017x over previous
import jax
import jax.numpy as jnp
from jax import lax
from jax.experimental import pallas as pl
from jax.experimental.pallas import tpu as pltpu
from jax.experimental.pallas import tpu_sc as plsc

TOTAL = 32768
NBINS = 16
NS = 16
LANES = 16
CHUNK = TOTAL // NS
SEARCH_STEPS = 12


def _sc_body(slot_hbm, nt_hbm, out_hbm, chunk_v, nt_v, hist_v, rows_v,
             rows_l, shift_v, sem_a, sem_b):
    sid = lax.axis_index("s")
    base = sid * CHUNK

    cp_chunk = pltpu.make_async_copy(
        slot_hbm.at[pl.ds(base, CHUNK)], chunk_v, sem_a)
    cp_chunk.start()

    @pl.when(sid == 0)
    def _():
        pltpu.make_async_copy(nt_hbm, nt_v, sem_b).start()

    cp_chunk.wait()

    zeros = jnp.zeros((LANES,), jnp.int32)
    lane_iota = lax.iota(jnp.int32, LANES)

    lo = zeros
    hi = jnp.full((LANES,), CHUNK, jnp.int32)
    target = lane_iota + 1
    cap = jnp.full((LANES,), CHUNK - 1, jnp.int32)
    for _ in range(SEARCH_STEPS):
        mid = jnp.minimum((lo + hi) >> 1, cap)
        c = plsc.load_gather(chunk_v, [mid])
        pred = c < target
        lo = jnp.where(pred, mid + 1, lo)
        hi = jnp.where(pred, hi, mid)

    hist_v[...] = lo
    pltpu.sync_copy(hist_v, rows_v.at[pl.ds(sid * NBINS, NBINS)])
    plsc.subcore_barrier()

    @pl.when(sid == 0)
    def _():
        pltpu.sync_copy(rows_v, rows_l)
        glb_hi = zeros
        for r in range(NS):
            glb_hi = glb_hi + rows_l[pl.ds(r * NBINS, NBINS)]
        shift_v[pl.ds(0, LANES)] = zeros
        plsc.store_scatter(shift_v, [lane_iota + 1], glb_hi)
        glb_lo = shift_v[pl.ds(0, LANES)]
        pltpu.make_async_copy(nt_hbm, nt_v, sem_b).wait()
        nt_vec = nt_v[...]
        hist_v[...] = (jnp.minimum(glb_hi, nt_vec)
                       - jnp.minimum(glb_lo, nt_vec))
        pltpu.sync_copy(hist_v, out_hbm)


@jax.jit
def _counts_sc(slot_ids, nt_vec):
    mesh = plsc.VectorSubcoreMesh(
        core_axis_name="c", subcore_axis_name="s", num_cores=1,
        num_subcores=NS)
    return pl.kernel(
        _sc_body,
        out_type=jax.ShapeDtypeStruct((NBINS,), jnp.int32),
        mesh=mesh,
        scratch_types=[
            pltpu.VMEM((CHUNK,), jnp.int32),
            pltpu.VMEM((LANES,), jnp.int32),
            pltpu.VMEM((NBINS,), jnp.int32),
            pltpu.VMEM_SHARED((NS * NBINS,), jnp.int32),
            pltpu.VMEM((NS * NBINS,), jnp.int32),
            pltpu.VMEM((LANES + 1,), jnp.int32),
            pltpu.SemaphoreType.DMA,
            pltpu.SemaphoreType.DMA,
        ],
        compiler_params=pltpu.CompilerParams(needs_layout_passes=False),
    )(slot_ids, nt_vec)


def kernel(tokens, slot_ids, pos_ids, num_tokens, max_slots):
    nt_vec = jnp.full((LANES,), num_tokens, dtype=jnp.int32)
    return _counts_sc(slot_ids, nt_vec)


# speedup vs baseline: 1.0481x; 1.0017x over previous
import jax
import jax.numpy as jnp
from jax import lax
from jax.experimental import pallas as pl
from jax.experimental.pallas import tpu as pltpu
from jax.experimental.pallas import tpu_sc as plsc

TOTAL = 32768
NBINS = 16
NS = 16
LANES = 16
CHUNK = TOTAL // NS
SEARCH_STEPS = 12


def _sc_body(slot_hbm, nt_hbm, out_hbm, chunk_v, nt_v, hist_v, rows_v,
             rows_l, sem_a, sem_b):
    sid = lax.axis_index("s")
    base = sid * CHUNK

    cp_chunk = pltpu.make_async_copy(
        slot_hbm.at[pl.ds(base, CHUNK)], chunk_v, sem_a)
    cp_chunk.start()

    @pl.when(sid == 0)
    def _():
        pltpu.make_async_copy(nt_hbm, nt_v, sem_b).start()

    cp_chunk.wait()

    zeros = jnp.zeros((LANES,), jnp.int32)
    lane_iota = lax.iota(jnp.int32, LANES)

    lo = zeros
    hi = jnp.full((LANES,), CHUNK, jnp.int32)
    target = lane_iota + 1
    cap = jnp.full((LANES,), CHUNK - 1, jnp.int32)
    for _ in range(SEARCH_STEPS):
        mid = jnp.minimum((lo + hi) >> 1, cap)
        c = plsc.load_gather(chunk_v, [mid])
        pred = c < target
        lo = jnp.where(pred, mid + 1, lo)
        hi = jnp.where(pred, hi, mid)

    hist_v[...] = lo
    pltpu.sync_copy(hist_v, rows_v.at[pl.ds(sid * NBINS, NBINS)])
    plsc.subcore_barrier()

    @pl.when(sid == 0)
    def _():
        pltpu.sync_copy(rows_v, rows_l)
        glb_hi = zeros
        for r in range(NS):
            glb_hi = glb_hi + rows_l[pl.ds(r * NBINS, NBINS)]
        sh_idx = jnp.maximum(lane_iota - 1, 0)
        shifted = lax.gather(
            glb_hi, sh_idx[:, None],
            lax.GatherDimensionNumbers(
                offset_dims=(), collapsed_slice_dims=(0,),
                start_index_map=(0,)),
            (1,), mode=lax.GatherScatterMode.PROMISE_IN_BOUNDS)
        glb_lo = jnp.where(lane_iota == 0, zeros, shifted)
        pltpu.make_async_copy(nt_hbm, nt_v, sem_b).wait()
        nt_vec = nt_v[...]
        hist_v[...] = (jnp.minimum(glb_hi, nt_vec)
                       - jnp.minimum(glb_lo, nt_vec))
        pltpu.sync_copy(hist_v, out_hbm)


@jax.jit
def _counts_sc(slot_ids, nt_vec):
    mesh = plsc.VectorSubcoreMesh(
        core_axis_name="c", subcore_axis_name="s", num_cores=1,
        num_subcores=NS)
    return pl.kernel(
        _sc_body,
        out_type=jax.ShapeDtypeStruct((NBINS,), jnp.int32),
        mesh=mesh,
        scratch_types=[
            pltpu.VMEM((CHUNK,), jnp.int32),
            pltpu.VMEM((LANES,), jnp.int32),
            pltpu.VMEM((NBINS,), jnp.int32),
            pltpu.VMEM_SHARED((NS * NBINS,), jnp.int32),
            pltpu.VMEM((NS * NBINS,), jnp.int32),
            pltpu.SemaphoreType.DMA,
            pltpu.SemaphoreType.DMA,
        ],
        compiler_params=pltpu.CompilerParams(needs_layout_passes=False),
    )(slot_ids, nt_vec)


def kernel(tokens, slot_ids, pos_ids, num_tokens, max_slots):
    nt_vec = jnp.full((LANES,), num_tokens, dtype=jnp.int32)
    return _counts_sc(slot_ids, nt_vec)
